# baseline (speedup 1.0000x reference)
import functools
import math

import jax
import jax.numpy as jnp
from jax import lax
from jax.experimental import pallas as pl
from jax.experimental.pallas import tpu as pltpu
from jax.experimental.pallas import tpu_sc as plsc

_B = 8
_L = 2048
_D = 32
_P = 32
_RP = 32
_PRED = 96
_H = 8
_DH = 4
_DFF = 128
_N = _B * _D


def _mask_body(x_ref, xm_ref, vm_ref, ms_ref):
    xm = xm_ref[...]
    vm_ref[...] = x_ref[...] * xm
    ms_ref[...] = jnp.sum(xm, axis=-1)


_NW = 32
_TPW = (_P * _B) // _NW
_VPAD = 128


def _sc_interp(time_hbm, msum_hbm, vm_hbm, refs_hbm, se_hbm,
               rep_hbm, so_hbm,
               time_v, msum_v, vmbuf, refbuf, sebuf, ebuf, rep_v, so_v):
    wid = lax.axis_index("s") * 2 + lax.axis_index("c")
    pltpu.sync_copy(se_hbm, sebuf.at[pl.ds(0, 2 * _P)])
    for k in range(_TPW):
        tau = wid + k * _NW
        p = tau // _B
        b = tau % _B
        start = sebuf[pl.ds(p, 16)][0]
        end = sebuf[pl.ds(_P + p, 16)][0]
        pltpu.sync_copy(time_hbm.at[b], time_v.at[pl.ds(0, _L)])
        pltpu.sync_copy(msum_hbm.at[b], msum_v.at[pl.ds(0, _L)])
        pltpu.sync_copy(refs_hbm.at[p], refbuf)
        rv0 = refbuf[pl.ds(0, 16)]
        rv1 = refbuf[pl.ds(16, 16)]

        def bs_lo(_, ab):
            a, bnd = ab
            mid = (a + bnd) // 2
            tm = time_v[pl.ds(mid, 16)][0]
            below = tm < start
            return jnp.where(below, mid + 1, a), jnp.where(below, bnd, mid)

        def bs_hi(_, ab):
            a, bnd = ab
            mid = (a + bnd) // 2
            tm = time_v[pl.ds(mid, 16)][0]
            below = tm <= end
            return jnp.where(below, mid + 1, a), jnp.where(below, bnd, mid)

        _, lo = lax.fori_loop(0, 12, bs_lo, (0, _L))
        _, hi = lax.fori_loop(0, 12, bs_hi, (0, _L))

        lo8 = pl.multiple_of((lo // 8) * 8, 8)
        nch = (hi - lo8 + _VPAD - 1) // _VPAD

        neg = jnp.full((16,), -1e30, jnp.float32)

        def max_body(l, carry):
            m0, m1 = carry
            t_l = time_v[pl.ds(l, 16)][0]
            obsf = jnp.where(msum_v[pl.ds(l, 16)][0] > 0.0, 1.0, 0.0)
            pen = (1.0 - obsf) * (-1e30)
            d0 = rv0 - t_l
            d1 = rv1 - t_l
            s0 = -(d0 * d0) * 0.125 + pen
            s1 = -(d1 * d1) * 0.125 + pen
            return jnp.maximum(m0, s0), jnp.maximum(m1, s1)

        m0, m1 = lax.fori_loop(lo, hi, max_body, (neg, neg))

        for r in range(_RP):
            rep_v[r, pl.ds(0, 16)] = jnp.zeros((16,), jnp.float32)
            rep_v[r, pl.ds(16, 16)] = jnp.zeros((16,), jnp.float32)

        def chunk_body(kk, carry):
            base = lo8 + kk * _VPAD
            pltpu.sync_copy(vm_hbm.at[b, pl.ds(base, _VPAD)], vmbuf)

            def acc_body(l, c2):
                so0, so1 = c2
                t_l = time_v[pl.ds(l, 16)][0]
                obsf = jnp.where(msum_v[pl.ds(l, 16)][0] > 0.0, 1.0, 0.0)
                d0 = rv0 - t_l
                d1 = rv1 - t_l
                e0 = jnp.exp(jnp.minimum(-(d0 * d0) * 0.125 - m0, 0.0)) * obsf
                e1 = jnp.exp(jnp.minimum(-(d1 * d1) * 0.125 - m1, 0.0)) * obsf
                ebuf[pl.ds(0, 16)] = e0
                ebuf[pl.ds(16, 16)] = e1
                row = l - base
                vr0 = vmbuf[row, pl.ds(0, 16)]
                vr1 = vmbuf[row, pl.ds(16, 16)]
                for r in range(_RP):
                    er = ebuf[pl.ds(r, 16)][0]
                    rep_v[r, pl.ds(0, 16)] = rep_v[r, pl.ds(0, 16)] + er * vr0
                    rep_v[r, pl.ds(16, 16)] = rep_v[r, pl.ds(16, 16)] + er * vr1
                return so0 + e0, so1 + e1

            p0 = jnp.maximum(lo, base)
            p1 = jnp.minimum(hi, base + _VPAD)
            return lax.fori_loop(p0, p1, acc_body, carry)

        z16f = jnp.zeros((16,), jnp.float32)
        so0, so1 = lax.fori_loop(0, nch, chunk_body, (z16f, z16f))
        so_v[pl.ds(0, 16)] = so0
        so_v[pl.ds(16, 16)] = so1
        pltpu.sync_copy(rep_v, rep_hbm.at[tau])
        pltpu.sync_copy(so_v, so_hbm.at[tau])


def _asm_body(rep_ref, so_ref, refs_ref, wenc_ref, benc_ref, pe_ref, out_ref):
    del refs_ref
    inv = 1.0 / jnp.maximum(so_ref[...][..., None], 1e-30)
    rep = rep_ref[...] * inv
    wenc = wenc_ref[...]
    cols = []
    for b in range(_B):
        cols.append(jnp.dot(rep[0, b], wenc,
                            preferred_element_type=jnp.float32))
    slab = jnp.concatenate(cols, axis=1)
    out_ref[...] = slab + benc_ref[...] + pe_ref[0]


def _ln_rows(v, g, b):
    mu = jnp.mean(v, axis=0, keepdims=True)
    var = jnp.mean((v - mu) ** 2, axis=0, keepdims=True)
    return (v - mu) / jnp.sqrt(var + 1e-5) * g + b


def _xform_body(tok_ref, wqkv_ref, bqkv_ref, wo_ref, bo_ref, g1_ref, be1_ref,
                w1_ref, b1_ref, w2_ref, b2_ref, g2_ref, be2_ref, gf_ref,
                bf_ref, wh_ref, bh_ref, out_ref, st_ref):
    i = pl.program_id(0)

    @pl.when(i == 0)
    def _():
        st_ref[...] = tok_ref[...]

    tm = st_ref[...]
    qkv = (jnp.dot(wqkv_ref[0], tm, preferred_element_type=jnp.float32)
           + bqkv_ref[0])
    q3 = qkv[0:_RP].reshape(_RP, _P, _N)
    k3 = qkv[_RP:2 * _RP].reshape(_RP, _P, _N)
    v3 = qkv[2 * _RP:3 * _RP].reshape(_RP, _P, _N)
    orows = []
    for h in range(_H):
        att = None
        for j in range(_DH):
            c = 4 * h + j
            term = q3[c][:, None, :] * k3[c][None, :, :]
            att = term if att is None else att + term
        att = att * 0.5
        mx = jnp.max(att, axis=1, keepdims=True)
        ex = jnp.exp(att - mx)
        att = ex / jnp.sum(ex, axis=1, keepdims=True)
        for j in range(_DH):
            c = 4 * h + j
            orows.append(jnp.sum(att * v3[c][None, :, :], axis=1))
    o3 = jnp.stack(orows, axis=0)
    om = o3.reshape(_RP, _P * _N)
    om = (jnp.dot(wo_ref[0], om, preferred_element_type=jnp.float32)
          + bo_ref[0])
    tm = _ln_rows(tm + om, g1_ref[0], be1_ref[0])
    y = jax.nn.gelu(
        jnp.dot(w1_ref[0], tm, preferred_element_type=jnp.float32)
        + b1_ref[0])
    y = (jnp.dot(w2_ref[0], y, preferred_element_type=jnp.float32)
         + b2_ref[0])
    tm = _ln_rows(tm + y, g2_ref[0], be2_ref[0])
    st_ref[...] = tm

    @pl.when(i == 2)
    def _():
        tmf = _ln_rows(tm, gf_ref[...], bf_ref[...])
        flat = tmf.reshape(_RP, _P, _N).reshape(_RP * _P, _N)
        out_ref[...] = (jnp.dot(wh_ref[...], flat,
                                preferred_element_type=jnp.float32)
                        + bh_ref[...])


def kernel(x, x_mark, x_mask, W_enc, b_enc, Wq, bq, Wk, bk, Wv, bv, Wo, bo,
           ln1_g, ln1_b, W1, b1, W2, b2, ln2_g, ln2_b, lnf_g, lnf_b,
           W_head, b_head):
    time = x_mark[:, :, 0]
    patch_range = jnp.linspace(0.0, float(_L), _P + 1)
    refs3 = jnp.linspace(0.0, float(_L), _P * _RP).reshape(_P, _RP, 1)
    se = jnp.stack([patch_range[:-1], patch_range[1:]], axis=0)

    pos = jnp.arange(_P, dtype=jnp.float32)[:, None]
    div = jnp.exp(jnp.arange(0, _RP, 2, dtype=jnp.float32)
                  * (-math.log(10000.0) / _RP))
    pe = jnp.zeros((_P, _RP), jnp.float32)
    pe = pe.at[:, 0::2].set(jnp.sin(pos * div)).at[:, 1::2].set(
        jnp.cos(pos * div))
    peT3 = pe[:, :, None]
    benc_t = jnp.tile(b_enc, _B)[None, :]

    WqkvT = jnp.concatenate(
        [jnp.swapaxes(Wq, 1, 2), jnp.swapaxes(Wk, 1, 2),
         jnp.swapaxes(Wv, 1, 2)], axis=1)
    bqkv = jnp.concatenate([bq, bk, bv], axis=1)[:, :, None]
    WoT = jnp.swapaxes(Wo, 1, 2)
    W1T = jnp.swapaxes(W1, 1, 2)
    W2T = jnp.swapaxes(W2, 1, 2)
    WheadT = W_head.T

    vm, msum = pl.pallas_call(
        _mask_body,
        out_shape=(jax.ShapeDtypeStruct((_B, _L, _D), jnp.float32),
                   jax.ShapeDtypeStruct((_B, _L), jnp.float32)),
    )(x, x_mask)

    refs = jnp.linspace(0.0, float(_L), _P * _RP).reshape(_P, _RP)
    vm_pad = jnp.pad(vm, ((0, 0), (0, _VPAD), (0, 0)))

    mesh = plsc.VectorSubcoreMesh(core_axis_name="c", subcore_axis_name="s")
    sc_fn = pl.kernel(
        mesh=mesh,
        out_type=(jax.ShapeDtypeStruct((_P * _B, _RP, _D), jnp.float32),
                  jax.ShapeDtypeStruct((_P * _B, _RP), jnp.float32)),
        scratch_types=[
            pltpu.VMEM((_L + 16,), jnp.float32),
            pltpu.VMEM((_L + 16,), jnp.float32),
            pltpu.VMEM((_VPAD, _D), jnp.float32),
            pltpu.VMEM((_RP,), jnp.float32),
            pltpu.VMEM((2 * _P + 16,), jnp.float32),
            pltpu.VMEM((_RP + 16,), jnp.float32),
            pltpu.VMEM((_RP, _D), jnp.float32),
            pltpu.VMEM((_RP,), jnp.float32),
        ])(_sc_interp)
    rep, so = sc_fn(time, msum, vm_pad, refs, se.reshape(-1))

    tok = pl.pallas_call(
        _asm_body,
        grid=(_P,),
        in_specs=[
            pl.BlockSpec((1, _B, _RP, _D), lambda p: (p, 0, 0, 0)),
            pl.BlockSpec((1, _B, _RP), lambda p: (p, 0, 0)),
            pl.BlockSpec((1, _RP, 1), lambda p: (p, 0, 0)),
            pl.BlockSpec((_D, _D), lambda p: (0, 0)),
            pl.BlockSpec((1, _N), lambda p: (0, 0)),
            pl.BlockSpec((1, _RP, 1), lambda p: (p, 0, 0)),
        ],
        out_specs=pl.BlockSpec((_RP, _N), lambda p: (0, p)),
        out_shape=jax.ShapeDtypeStruct((_RP, _P * _N), jnp.float32),
    )(rep.reshape(_P, _B, _RP, _D), so.reshape(_P, _B, _RP), refs3, W_enc,
      benc_t, peT3)

    lw = lambda shp: pl.BlockSpec(
        (1,) + shp, lambda i: (i,) + tuple(0 for _ in shp))
    cw = lambda shp: pl.BlockSpec(shp, lambda i: tuple(0 for _ in shp))
    out2 = pl.pallas_call(
        _xform_body,
        grid=(3,),
        in_specs=[
            cw((_RP, _P * _N)),
            lw((3 * _RP, _RP)), lw((3 * _RP, 1)),
            lw((_RP, _RP)), lw((_RP, 1)),
            lw((_RP, 1)), lw((_RP, 1)),
            lw((_DFF, _RP)), lw((_DFF, 1)),
            lw((_RP, _DFF)), lw((_RP, 1)),
            lw((_RP, 1)), lw((_RP, 1)),
            cw((_RP, 1)), cw((_RP, 1)),
            cw((_PRED, _RP * _P)), cw((_PRED, 1)),
        ],
        out_specs=cw((_PRED, _N)),
        out_shape=jax.ShapeDtypeStruct((_PRED, _N), jnp.float32),
        scratch_shapes=[pltpu.VMEM((_RP, _P * _N), jnp.float32)],
    )(tok, WqkvT, bqkv, WoT, bo[:, :, None], ln1_g[:, :, None],
      ln1_b[:, :, None], W1T, b1[:, :, None], W2T, b2[:, :, None],
      ln2_g[:, :, None], ln2_b[:, :, None], lnf_g[:, None], lnf_b[:, None],
      WheadT, b_head[:, None])

    return out2.reshape(_PRED, _B, _D).transpose(1, 0, 2)


# speedup vs baseline: 1.0641x; 1.0641x over previous
import functools
import math

import jax
import jax.numpy as jnp
from jax import lax
from jax.experimental import pallas as pl
from jax.experimental.pallas import tpu as pltpu
from jax.experimental.pallas import tpu_sc as plsc

_B = 8
_L = 2048
_D = 32
_P = 32
_RP = 32
_PRED = 96
_H = 8
_DH = 4
_DFF = 128
_N = _B * _D


def _mask_body(x_ref, xm_ref, vm_ref, ms_ref):
    xm = xm_ref[...]
    vm_ref[...] = x_ref[...] * xm
    ms_ref[...] = jnp.sum(xm, axis=-1)


_NW = 32
_TPW = (_P * _B) // _NW
_VPAD = 128


def _sc_interp(time_hbm, msum_hbm, vm_hbm, refs_hbm, se_hbm,
               rep_hbm, so_hbm,
               time_v, msum_v, vmbuf, refbuf, sebuf, ebuf, rep_v, so_v):
    wid = lax.axis_index("s") * 2 + lax.axis_index("c")
    pltpu.sync_copy(se_hbm, sebuf.at[pl.ds(0, 2 * _P)])
    b = wid % _B
    pltpu.sync_copy(time_hbm.at[b], time_v.at[pl.ds(0, _L)])
    pltpu.sync_copy(msum_hbm.at[b], msum_v.at[pl.ds(0, _L)])
    for k in range(_TPW):
        tau = wid + k * _NW
        p = tau // _B
        start = sebuf[pl.ds(p, 16)][0]
        end = sebuf[pl.ds(_P + p, 16)][0]
        pltpu.sync_copy(refs_hbm.at[p], refbuf)
        rv0 = refbuf[pl.ds(0, 16)]
        rv1 = refbuf[pl.ds(16, 16)]

        def bs_lo(_, ab):
            a, bnd = ab
            mid = (a + bnd) // 2
            tm = time_v[pl.ds(mid, 16)][0]
            below = tm < start
            return jnp.where(below, mid + 1, a), jnp.where(below, bnd, mid)

        def bs_hi(_, ab):
            a, bnd = ab
            mid = (a + bnd) // 2
            tm = time_v[pl.ds(mid, 16)][0]
            below = tm <= end
            return jnp.where(below, mid + 1, a), jnp.where(below, bnd, mid)

        _, lo = lax.fori_loop(0, 12, bs_lo, (0, _L))
        _, hi = lax.fori_loop(0, 12, bs_hi, (0, _L))

        lo8 = pl.multiple_of((lo // 8) * 8, 8)
        nch = (hi - lo8 + _VPAD - 1) // _VPAD

        neg = jnp.full((16,), -1e30, jnp.float32)

        def max_body(l, carry):
            m0, m1 = carry
            t_l = time_v[pl.ds(l, 16)][0]
            obsf = jnp.where(msum_v[pl.ds(l, 16)][0] > 0.0, 1.0, 0.0)
            pen = (1.0 - obsf) * (-1e30)
            d0 = rv0 - t_l
            d1 = rv1 - t_l
            s0 = -(d0 * d0) * 0.125 + pen
            s1 = -(d1 * d1) * 0.125 + pen
            return jnp.maximum(m0, s0), jnp.maximum(m1, s1)

        m0, m1 = lax.fori_loop(lo, hi, max_body, (neg, neg))

        for r in range(_RP):
            rep_v[r, pl.ds(0, 16)] = jnp.zeros((16,), jnp.float32)
            rep_v[r, pl.ds(16, 16)] = jnp.zeros((16,), jnp.float32)

        def chunk_body(kk, carry):
            base = lo8 + kk * _VPAD
            pltpu.sync_copy(vm_hbm.at[b, pl.ds(base, _VPAD)], vmbuf)

            def acc_body(l, c2):
                so0, so1 = c2
                t_l = time_v[pl.ds(l, 16)][0]
                obsf = jnp.where(msum_v[pl.ds(l, 16)][0] > 0.0, 1.0, 0.0)
                d0 = rv0 - t_l
                d1 = rv1 - t_l
                e0 = jnp.exp(jnp.minimum(-(d0 * d0) * 0.125 - m0, 0.0)) * obsf
                e1 = jnp.exp(jnp.minimum(-(d1 * d1) * 0.125 - m1, 0.0)) * obsf
                ebuf[pl.ds(0, 16)] = e0
                ebuf[pl.ds(16, 16)] = e1
                row = l - base
                vr0 = vmbuf[row, pl.ds(0, 16)]
                vr1 = vmbuf[row, pl.ds(16, 16)]
                for r in range(_RP):
                    er = ebuf[pl.ds(r, 16)][0]
                    plsc.addupdate(rep_v.at[r, pl.ds(0, 16)], er * vr0)
                    plsc.addupdate(rep_v.at[r, pl.ds(16, 16)], er * vr1)
                return so0 + e0, so1 + e1

            p0 = jnp.maximum(lo, base)
            p1 = jnp.minimum(hi, base + _VPAD)
            return lax.fori_loop(p0, p1, acc_body, carry)

        z16f = jnp.zeros((16,), jnp.float32)
        so0, so1 = lax.fori_loop(0, nch, chunk_body, (z16f, z16f))
        so_v[pl.ds(0, 16)] = so0
        so_v[pl.ds(16, 16)] = so1
        pltpu.sync_copy(rep_v, rep_hbm.at[tau])
        pltpu.sync_copy(so_v, so_hbm.at[tau])


def _asm_body(rep_ref, so_ref, refs_ref, wenc_ref, benc_ref, pe_ref, out_ref):
    del refs_ref
    inv = 1.0 / jnp.maximum(so_ref[...][..., None], 1e-30)
    rep = rep_ref[...] * inv
    wenc = wenc_ref[...]
    cols = []
    for b in range(_B):
        cols.append(jnp.dot(rep[0, b], wenc,
                            preferred_element_type=jnp.float32))
    slab = jnp.concatenate(cols, axis=1)
    out_ref[...] = slab + benc_ref[...] + pe_ref[0]


def _ln_rows(v, g, b):
    mu = jnp.mean(v, axis=0, keepdims=True)
    var = jnp.mean((v - mu) ** 2, axis=0, keepdims=True)
    return (v - mu) / jnp.sqrt(var + 1e-5) * g + b


def _xform_body(tok_ref, wqkv_ref, bqkv_ref, wo_ref, bo_ref, g1_ref, be1_ref,
                w1_ref, b1_ref, w2_ref, b2_ref, g2_ref, be2_ref, gf_ref,
                bf_ref, wh_ref, bh_ref, out_ref, st_ref):
    i = pl.program_id(0)

    @pl.when(i == 0)
    def _():
        st_ref[...] = tok_ref[...]

    tm = st_ref[...]
    qkv = (jnp.dot(wqkv_ref[0], tm, preferred_element_type=jnp.float32)
           + bqkv_ref[0])
    q3 = qkv[0:_RP].reshape(_RP, _P, _N)
    k3 = qkv[_RP:2 * _RP].reshape(_RP, _P, _N)
    v3 = qkv[2 * _RP:3 * _RP].reshape(_RP, _P, _N)
    orows = []
    for h in range(_H):
        att = None
        for j in range(_DH):
            c = 4 * h + j
            term = q3[c][:, None, :] * k3[c][None, :, :]
            att = term if att is None else att + term
        att = att * 0.5
        mx = jnp.max(att, axis=1, keepdims=True)
        ex = jnp.exp(att - mx)
        att = ex / jnp.sum(ex, axis=1, keepdims=True)
        for j in range(_DH):
            c = 4 * h + j
            orows.append(jnp.sum(att * v3[c][None, :, :], axis=1))
    o3 = jnp.stack(orows, axis=0)
    om = o3.reshape(_RP, _P * _N)
    om = (jnp.dot(wo_ref[0], om, preferred_element_type=jnp.float32)
          + bo_ref[0])
    tm = _ln_rows(tm + om, g1_ref[0], be1_ref[0])
    y = jax.nn.gelu(
        jnp.dot(w1_ref[0], tm, preferred_element_type=jnp.float32)
        + b1_ref[0])
    y = (jnp.dot(w2_ref[0], y, preferred_element_type=jnp.float32)
         + b2_ref[0])
    tm = _ln_rows(tm + y, g2_ref[0], be2_ref[0])
    st_ref[...] = tm

    @pl.when(i == 2)
    def _():
        tmf = _ln_rows(tm, gf_ref[...], bf_ref[...])
        flat = tmf.reshape(_RP, _P, _N).reshape(_RP * _P, _N)
        out_ref[...] = (jnp.dot(wh_ref[...], flat,
                                preferred_element_type=jnp.float32)
                        + bh_ref[...])


def kernel(x, x_mark, x_mask, W_enc, b_enc, Wq, bq, Wk, bk, Wv, bv, Wo, bo,
           ln1_g, ln1_b, W1, b1, W2, b2, ln2_g, ln2_b, lnf_g, lnf_b,
           W_head, b_head):
    time = x_mark[:, :, 0]
    patch_range = jnp.linspace(0.0, float(_L), _P + 1)
    refs3 = jnp.linspace(0.0, float(_L), _P * _RP).reshape(_P, _RP, 1)
    se = jnp.stack([patch_range[:-1], patch_range[1:]], axis=0)

    pos = jnp.arange(_P, dtype=jnp.float32)[:, None]
    div = jnp.exp(jnp.arange(0, _RP, 2, dtype=jnp.float32)
                  * (-math.log(10000.0) / _RP))
    pe = jnp.zeros((_P, _RP), jnp.float32)
    pe = pe.at[:, 0::2].set(jnp.sin(pos * div)).at[:, 1::2].set(
        jnp.cos(pos * div))
    peT3 = pe[:, :, None]
    benc_t = jnp.tile(b_enc, _B)[None, :]

    WqkvT = jnp.concatenate(
        [jnp.swapaxes(Wq, 1, 2), jnp.swapaxes(Wk, 1, 2),
         jnp.swapaxes(Wv, 1, 2)], axis=1)
    bqkv = jnp.concatenate([bq, bk, bv], axis=1)[:, :, None]
    WoT = jnp.swapaxes(Wo, 1, 2)
    W1T = jnp.swapaxes(W1, 1, 2)
    W2T = jnp.swapaxes(W2, 1, 2)
    WheadT = W_head.T

    vm, msum = pl.pallas_call(
        _mask_body,
        out_shape=(jax.ShapeDtypeStruct((_B, _L, _D), jnp.float32),
                   jax.ShapeDtypeStruct((_B, _L), jnp.float32)),
    )(x, x_mask)

    refs = jnp.linspace(0.0, float(_L), _P * _RP).reshape(_P, _RP)
    vm_pad = jnp.pad(vm, ((0, 0), (0, _VPAD), (0, 0)))

    mesh = plsc.VectorSubcoreMesh(core_axis_name="c", subcore_axis_name="s")
    sc_fn = pl.kernel(
        mesh=mesh,
        out_type=(jax.ShapeDtypeStruct((_P * _B, _RP, _D), jnp.float32),
                  jax.ShapeDtypeStruct((_P * _B, _RP), jnp.float32)),
        scratch_types=[
            pltpu.VMEM((_L + 16,), jnp.float32),
            pltpu.VMEM((_L + 16,), jnp.float32),
            pltpu.VMEM((_VPAD, _D), jnp.float32),
            pltpu.VMEM((_RP,), jnp.float32),
            pltpu.VMEM((2 * _P + 16,), jnp.float32),
            pltpu.VMEM((_RP + 16,), jnp.float32),
            pltpu.VMEM((_RP, _D), jnp.float32),
            pltpu.VMEM((_RP,), jnp.float32),
        ])(_sc_interp)
    rep, so = sc_fn(time, msum, vm_pad, refs, se.reshape(-1))

    tok = pl.pallas_call(
        _asm_body,
        grid=(_P,),
        in_specs=[
            pl.BlockSpec((1, _B, _RP, _D), lambda p: (p, 0, 0, 0)),
            pl.BlockSpec((1, _B, _RP), lambda p: (p, 0, 0)),
            pl.BlockSpec((1, _RP, 1), lambda p: (p, 0, 0)),
            pl.BlockSpec((_D, _D), lambda p: (0, 0)),
            pl.BlockSpec((1, _N), lambda p: (0, 0)),
            pl.BlockSpec((1, _RP, 1), lambda p: (p, 0, 0)),
        ],
        out_specs=pl.BlockSpec((_RP, _N), lambda p: (0, p)),
        out_shape=jax.ShapeDtypeStruct((_RP, _P * _N), jnp.float32),
    )(rep.reshape(_P, _B, _RP, _D), so.reshape(_P, _B, _RP), refs3, W_enc,
      benc_t, peT3)

    lw = lambda shp: pl.BlockSpec(
        (1,) + shp, lambda i: (i,) + tuple(0 for _ in shp))
    cw = lambda shp: pl.BlockSpec(shp, lambda i: tuple(0 for _ in shp))
    out2 = pl.pallas_call(
        _xform_body,
        grid=(3,),
        in_specs=[
            cw((_RP, _P * _N)),
            lw((3 * _RP, _RP)), lw((3 * _RP, 1)),
            lw((_RP, _RP)), lw((_RP, 1)),
            lw((_RP, 1)), lw((_RP, 1)),
            lw((_DFF, _RP)), lw((_DFF, 1)),
            lw((_RP, _DFF)), lw((_RP, 1)),
            lw((_RP, 1)), lw((_RP, 1)),
            cw((_RP, 1)), cw((_RP, 1)),
            cw((_PRED, _RP * _P)), cw((_PRED, 1)),
        ],
        out_specs=cw((_PRED, _N)),
        out_shape=jax.ShapeDtypeStruct((_PRED, _N), jnp.float32),
        scratch_shapes=[pltpu.VMEM((_RP, _P * _N), jnp.float32)],
    )(tok, WqkvT, bqkv, WoT, bo[:, :, None], ln1_g[:, :, None],
      ln1_b[:, :, None], W1T, b1[:, :, None], W2T, b2[:, :, None],
      ln2_g[:, :, None], ln2_b[:, :, None], lnf_g[:, None], lnf_b[:, None],
      WheadT, b_head[:, None])

    return out2.reshape(_PRED, _B, _D).transpose(1, 0, 2)
